# baseline (speedup 1.0000x reference)
import functools

import jax
import jax.numpy as jnp
from jax import lax
from jax.experimental import pallas as pl
from jax.experimental.pallas import tpu as pltpu
from jax.experimental.pallas import tpu_sc as plsc

_NC = 2
_NS = 16
_NW = _NC * _NS
_L = 16

_C = 8
_NBUF = 2


def _sc_add_body(tokens_hbm, t16_hbm, emb_hbm, out_hbm,
                 emb_v, t_v, row_v,
                 in0, in1, out0, out1,
                 sem_in0, sem_in1, sem_out0, sem_out1):
    R, D = tokens_hbm.shape
    rows_per_w = R // _NW
    nchunks = rows_per_w // _C

    wid = lax.axis_index("s") * _NC + lax.axis_index("c")
    base = wid * rows_per_w

    pltpu.sync_copy(emb_hbm, emb_v)
    pltpu.sync_copy(t16_hbm, t_v)
    tvec = t_v[...]
    is_row0 = tvec == 0
    for j in range(D // _L):
        sl = pl.ds(j * _L, _L)
        row_v[sl] = jnp.where(is_row0, emb_v[0, sl], emb_v[1, sl])

    in_bufs = (in0, in1)
    out_bufs = (out0, out1)
    sems_in = (sem_in0, sem_in1)
    sems_out = (sem_out0, sem_out1)

    def start_in(c, b):
        pltpu.make_async_copy(
            tokens_hbm.at[pl.ds(base + c * _C, _C)], in_bufs[b], sems_in[b]
        ).start()

    def wait_in(b):
        pltpu.make_async_copy(
            tokens_hbm.at[pl.ds(base, _C)], in_bufs[b], sems_in[b]
        ).wait()

    def start_out(c, b):
        pltpu.make_async_copy(
            out_bufs[b], out_hbm.at[pl.ds(base + c * _C, _C)], sems_out[b]
        ).start()

    def wait_out(b):
        pltpu.make_async_copy(
            out_bufs[b], out_hbm.at[pl.ds(base, _C)], sems_out[b]
        ).wait()

    start_in(0, 0)
    start_in(1, 1)

    def round_body(g, _):
        for b in range(_NBUF):
            c = g * _NBUF + b
            wait_in(b)

            @pl.when(g >= 1)
            def _():
                wait_out(b)

            ib, ob = in_bufs[b], out_bufs[b]

            @plsc.parallel_loop(0, D // _L, unroll=16)
            def _(j):
                sl = pl.ds(j * _L, _L)
                rv = row_v[sl]
                for r in range(_C):
                    ob[r, sl] = ib[r, sl] + rv

            @pl.when(g + 1 < nchunks // _NBUF)
            def _():
                start_in(c + _NBUF, b)

            start_out(c, b)
        return 0

    lax.fori_loop(0, nchunks // _NBUF, round_body, 0)

    wait_out(0)
    wait_out(1)


def kernel(tokens, t, emb):
    B, S, D = tokens.shape
    R = B * S
    flat = tokens.reshape(R, D)
    t16 = jnp.full((_L,), jnp.asarray(t, jnp.int32))

    mesh = plsc.VectorSubcoreMesh(core_axis_name="c", subcore_axis_name="s")
    run = pl.kernel(
        _sc_add_body,
        out_type=jax.ShapeDtypeStruct((R, D), tokens.dtype),
        mesh=mesh,
        scratch_types=[
            pltpu.VMEM((emb.shape[0], D), jnp.float32),
            pltpu.VMEM((_L,), jnp.int32),
            pltpu.VMEM((D,), jnp.float32),
            pltpu.VMEM((_C, D), jnp.float32),
            pltpu.VMEM((_C, D), jnp.float32),
            pltpu.VMEM((_C, D), jnp.float32),
            pltpu.VMEM((_C, D), jnp.float32),
            pltpu.SemaphoreType.DMA,
            pltpu.SemaphoreType.DMA,
            pltpu.SemaphoreType.DMA,
            pltpu.SemaphoreType.DMA,
        ],
    )
    out = run(flat, t16, emb)
    return out.reshape(B, S, D)


# speedup vs baseline: 1.1364x; 1.1364x over previous
import jax
import jax.numpy as jnp
from jax import lax
from jax.experimental import pallas as pl
from jax.experimental.pallas import tpu as pltpu
from jax.experimental.pallas import tpu_sc as plsc

_NC = 2
_L = 16


def _sc_lookup_body(t16_hbm, emb_hbm, row_hbm, emb_v, t_v, row_v):
    wid = lax.axis_index("s") * _NC + lax.axis_index("c")

    @pl.when(wid == 0)
    def _():
        K, D = emb_hbm.shape
        pltpu.sync_copy(emb_hbm, emb_v)
        pltpu.sync_copy(t16_hbm, t_v)
        is_row0 = t_v[...] == 0
        for j in range(D // _L):
            sl = pl.ds(j * _L, _L)
            row_v[sl] = jnp.where(is_row0, emb_v[0, sl], emb_v[1, sl])
        pltpu.sync_copy(row_v, row_hbm)


def _tc_add_body(x_ref, row_ref, o_ref):
    o_ref[...] = x_ref[...] + row_ref[...].reshape(1, -1)


def kernel(tokens, t, emb):
    B, S, D = tokens.shape
    R = B * S
    flat = tokens.reshape(R, D)
    t16 = jnp.full((_L,), jnp.asarray(t, jnp.int32))

    mesh = plsc.VectorSubcoreMesh(core_axis_name="c", subcore_axis_name="s")
    lookup = pl.kernel(
        _sc_lookup_body,
        out_type=jax.ShapeDtypeStruct((D,), emb.dtype),
        mesh=mesh,
        scratch_types=[
            pltpu.VMEM((emb.shape[0], D), jnp.float32),
            pltpu.VMEM((_L,), jnp.int32),
            pltpu.VMEM((D,), jnp.float32),
        ],
    )
    row = lookup(t16, emb)

    BLK = 1024
    out = pl.pallas_call(
        _tc_add_body,
        grid=(R // BLK,),
        in_specs=[
            pl.BlockSpec((BLK, D), lambda i: (i, 0)),
            pl.BlockSpec((D,), lambda i: (0,)),
        ],
        out_specs=pl.BlockSpec((BLK, D), lambda i: (i, 0)),
        out_shape=jax.ShapeDtypeStruct((R, D), tokens.dtype),
    )(flat, row)
    return out.reshape(B, S, D)
